# baseline (speedup 1.0000x reference)
import functools

import jax
import jax.numpy as jnp
from jax.experimental import pallas as pl
from jax.experimental.pallas import tpu as pltpu

_ROWS = 250_000
_BLK = 2_000
_GRID = _ROWS // _BLK


def _norm_sum_body(u_ref, v_ref, o_ref):
    step = pl.program_id(0)

    def block_total(x):
        x2 = x * x
        acc = jnp.zeros((x.shape[0], 1), jnp.float32)
        total = 0.0
        for k in range(4):
            s = jnp.sum(x2[:, k * 32:(k + 1) * 32], axis=1, keepdims=True)
            acc = acc + jnp.sqrt(s)
        return jnp.sum(acc)

    part = block_total(u_ref[...]) + block_total(v_ref[...])

    @pl.when(step == 0)
    def _init():
        o_ref[0, 0] = 0.0

    o_ref[0, 0] += part


def kernel(user_embedding, item_embedding):
    u = user_embedding.reshape(_ROWS, 128)
    v = item_embedding.reshape(_ROWS, 128)
    out = pl.pallas_call(
        _norm_sum_body,
        grid=(_GRID,),
        in_specs=[
            pl.BlockSpec((_BLK, 128), lambda i: (i, 0)),
            pl.BlockSpec((_BLK, 128), lambda i: (i, 0)),
        ],
        out_specs=pl.BlockSpec(memory_space=pltpu.SMEM),
        out_shape=jax.ShapeDtypeStruct((1, 1), jnp.float32),
    )(u, v)
    return out[0, 0]


# speedup vs baseline: 1.5676x; 1.5676x over previous
import jax
import jax.numpy as jnp
from jax.experimental import pallas as pl
from jax.experimental.pallas import tpu as pltpu

_ROWS = 250_000
_BLK = 10_000
_GRID = _ROWS // _BLK


def _norm_sum_body(u_ref, v_ref, o_ref):
    step = pl.program_id(0)

    lane = jax.lax.broadcasted_iota(jnp.int32, (128, 4), 0)
    seg = jax.lax.broadcasted_iota(jnp.int32, (128, 4), 1)
    S = (lane // 32 == seg).astype(jnp.float32)

    def block_total(x):
        x2 = x * x
        n2 = jax.lax.dot_general(
            x2, S, (((1,), (0,)), ((), ())),
            preferred_element_type=jnp.float32)
        return jnp.sum(jnp.sqrt(n2))

    part = block_total(u_ref[...]) + block_total(v_ref[...])

    @pl.when(step == 0)
    def _init():
        o_ref[0, 0] = 0.0

    o_ref[0, 0] += part


def kernel(user_embedding, item_embedding):
    u = user_embedding.reshape(_ROWS, 128)
    v = item_embedding.reshape(_ROWS, 128)
    out = pl.pallas_call(
        _norm_sum_body,
        grid=(_GRID,),
        in_specs=[
            pl.BlockSpec((_BLK, 128), lambda i: (i, 0)),
            pl.BlockSpec((_BLK, 128), lambda i: (i, 0)),
        ],
        out_specs=pl.BlockSpec(memory_space=pltpu.SMEM),
        out_shape=jax.ShapeDtypeStruct((1, 1), jnp.float32),
    )(u, v)
    return out[0, 0]


# speedup vs baseline: 1.6291x; 1.0392x over previous
import jax
import jax.numpy as jnp
from jax.experimental import pallas as pl
from jax.experimental.pallas import tpu as pltpu

_N = 1_000_000
_BLK = 8_000
_GRID = _N // _BLK


def _norm_sum_body(u_ref, v_ref, o_ref):
    step = pl.program_id(0)

    ones_col = jnp.ones((32, 1), jnp.float32)
    ones_row = jnp.ones((1, _BLK), jnp.float32)

    def block_total(x):
        x2 = x * x
        n2 = jax.lax.dot_general(
            x2, ones_col, (((1,), (0,)), ((), ())),
            preferred_element_type=jnp.float32)
        s = jnp.sqrt(n2)
        tot = jax.lax.dot_general(
            ones_row, s, (((1,), (0,)), ((), ())),
            preferred_element_type=jnp.float32)
        return tot[0, 0]

    part = block_total(u_ref[...]) + block_total(v_ref[...])

    @pl.when(step == 0)
    def _init():
        o_ref[0, 0] = 0.0

    o_ref[0, 0] += part


def kernel(user_embedding, item_embedding):
    out = pl.pallas_call(
        _norm_sum_body,
        grid=(_GRID,),
        in_specs=[
            pl.BlockSpec((_BLK, 32), lambda i: (i, 0)),
            pl.BlockSpec((_BLK, 32), lambda i: (i, 0)),
        ],
        out_specs=pl.BlockSpec(memory_space=pltpu.SMEM),
        out_shape=jax.ShapeDtypeStruct((1, 1), jnp.float32),
    )(user_embedding, item_embedding)
    return out[0, 0]
